# Initial kernel scaffold; baseline (speedup 1.0000x reference)
import jax
import jax.numpy as jnp
from jax.experimental import pallas as pl


def kernel(logits, illegal_action_masks, curr_positions, correlation_params):
    raise NotImplementedError("write your pallas kernel here")


# speedup vs baseline: 386.1269x; 386.1269x over previous
import functools

import jax
import jax.numpy as jnp
from jax.experimental import pallas as pl

ITERATIONS = 3
ACTION_DIMS = 5
FOV = 11
MAX_DIST = FOV // 2
NPARAM = FOV * FOV
AD2 = ACTION_DIMS * ACTION_DIMS


def _mf_kernel(logits_ref, ill_ref, pos_ref, post_ref, q0_ref, c2_ref, out_ref):
    A = logits_ref.shape[1]

    pos = pos_ref[0]
    post = post_ref[0]
    yc = pos[:, 0:1]
    xc = pos[:, 1:2]
    yr = post[0:1, :]
    xr = post[1:2, :]

    dy = jnp.abs(yc - yr)
    dx = jnp.abs(xc - xr)
    neighbor = jnp.maximum(dy, dx) <= MAX_DIST
    pidx = (dy + MAX_DIST) * FOV + (dx + MAX_DIST)
    pidx = jnp.where(neighbor, pidx, NPARAM)

    k_iota = jax.lax.broadcasted_iota(jnp.int32, (A, A, NPARAM), 2)
    oh = (pidx[:, :, None] == k_iota).astype(jnp.float32)
    oh2 = oh.reshape(A * A, NPARAM)
    we = jnp.dot(oh2, c2_ref[...], preferred_element_type=jnp.float32)
    we3 = we.reshape(A, A, AD2)

    ti = jax.lax.broadcasted_iota(jnp.int32, (ACTION_DIMS, AD2), 0)
    tc = jax.lax.broadcasted_iota(jnp.int32, (ACTION_DIMS, AD2), 1)
    texp = (tc - (tc // ACTION_DIMS) * ACTION_DIMS == ti).astype(jnp.float32)
    gi = jax.lax.broadcasted_iota(jnp.int32, (AD2, ACTION_DIMS), 0)
    gc = jax.lax.broadcasted_iota(jnp.int32, (AD2, ACTION_DIMS), 1)
    g = (gi // ACTION_DIMS == gc).astype(jnp.float32)

    lg = logits_ref[0]
    ill = ill_ref[0] * 1e10
    q_logits = q0_ref[0] - ill

    for _ in range(ITERATIONS):
        qp = jax.nn.softmax(q_logits, axis=-1)
        qtile = jnp.dot(qp, texp, preferred_element_type=jnp.float32)
        s = jnp.sum(we3 * qtile[None, :, :], axis=1)
        corr = jnp.dot(s, g, preferred_element_type=jnp.float32)
        q_logits = lg + corr - ill

    out_ref[0] = q_logits


@functools.partial(jax.jit, static_argnames=("interpret",))
def kernel(logits, illegal_action_masks, curr_positions, correlation_params,
           interpret: bool = False):
    B, A, AD = logits.shape
    pos32 = curr_positions.astype(jnp.int32)
    post32 = jnp.swapaxes(pos32, 1, 2)
    c2 = correlation_params.reshape(NPARAM, AD2)
    q0 = jax.random.uniform(jax.random.key(1), logits.shape, dtype=logits.dtype)

    grid = (B,)
    out = pl.pallas_call(
        _mf_kernel,
        grid=grid,
        in_specs=[
            pl.BlockSpec((1, A, AD), lambda b: (b, 0, 0)),
            pl.BlockSpec((1, A, AD), lambda b: (b, 0, 0)),
            pl.BlockSpec((1, A, 2), lambda b: (b, 0, 0)),
            pl.BlockSpec((1, 2, A), lambda b: (b, 0, 0)),
            pl.BlockSpec((1, A, AD), lambda b: (b, 0, 0)),
            pl.BlockSpec((NPARAM, AD2), lambda b: (0, 0)),
        ],
        out_specs=pl.BlockSpec((1, A, AD), lambda b: (b, 0, 0)),
        out_shape=jax.ShapeDtypeStruct((B, A, AD), logits.dtype),
        interpret=interpret,
    )(logits, illegal_action_masks, pos32, post32, q0, c2)
    return out


# speedup vs baseline: 794.9753x; 2.0588x over previous
import functools

import jax
import jax.numpy as jnp
from jax.experimental import pallas as pl

ITERATIONS = 3
ACTION_DIMS = 5
FOV = 11
MAX_DIST = FOV // 2
NPARAM = FOV * FOV
AD2 = ACTION_DIMS * ACTION_DIMS


def _mf_kernel(lgt_ref, illt_ref, pos_ref, post_ref, q0t_ref, c2t_ref, out_ref):
    A = lgt_ref.shape[2]

    pos = pos_ref[0]
    post = post_ref[0]
    yc = pos[:, 0:1]
    xc = pos[:, 1:2]
    yr = post[0:1, :]
    xr = post[1:2, :]

    dy = jnp.abs(yc - yr)
    dx = jnp.abs(xc - xr)
    neighbor = jnp.maximum(dy, dx) <= MAX_DIST
    pidx = (dy + MAX_DIST) * FOV + (dx + MAX_DIST)
    pidx = jnp.where(neighbor, pidx, NPARAM)

    src = jnp.broadcast_to(c2t_ref[...][:, None, :], (AD2, A, A)).reshape(AD2 * A, A)
    idx = jnp.broadcast_to(pidx[None, :, :], (AD2, A, A)).reshape(AD2 * A, A)
    wes = jnp.take_along_axis(src, idx, axis=1).reshape(AD2, A, A)

    ti = jax.lax.broadcasted_iota(jnp.int32, (AD2, ACTION_DIMS), 0)
    tc = jax.lax.broadcasted_iota(jnp.int32, (AD2, ACTION_DIMS), 1)
    texp2 = (ti - (ti // ACTION_DIMS) * ACTION_DIMS == tc).astype(jnp.float32)
    gi = jax.lax.broadcasted_iota(jnp.int32, (ACTION_DIMS, AD2), 0)
    gc = jax.lax.broadcasted_iota(jnp.int32, (ACTION_DIMS, AD2), 1)
    gt = (gc // ACTION_DIMS == gi).astype(jnp.float32)

    lgt = lgt_ref[0]
    illt = illt_ref[0] * 1e10
    qt = q0t_ref[0] - illt

    for _ in range(ITERATIONS):
        qpt = jax.nn.softmax(qt, axis=0)
        qtt = jnp.dot(texp2, qpt, preferred_element_type=jnp.float32)
        s2 = jnp.sum(wes * qtt[:, :, None], axis=1)
        corrt = jnp.dot(gt, s2, preferred_element_type=jnp.float32)
        qt = lgt + corrt - illt

    out_ref[0] = qt


@functools.partial(jax.jit, static_argnames=("interpret",))
def kernel(logits, illegal_action_masks, curr_positions, correlation_params,
           interpret: bool = False):
    B, A, AD = logits.shape
    pos32 = curr_positions.astype(jnp.int32)
    post32 = jnp.swapaxes(pos32, 1, 2)
    c2t = jnp.zeros((AD2, 128), correlation_params.dtype)
    c2t = c2t.at[:, :NPARAM].set(correlation_params.reshape(NPARAM, AD2).T)
    q0 = jax.random.uniform(jax.random.key(1), logits.shape, dtype=logits.dtype)

    lgt = jnp.swapaxes(logits, 1, 2)
    illt = jnp.swapaxes(illegal_action_masks, 1, 2)
    q0t = jnp.swapaxes(q0, 1, 2)

    grid = (B,)
    outt = pl.pallas_call(
        _mf_kernel,
        grid=grid,
        in_specs=[
            pl.BlockSpec((1, AD, A), lambda b: (b, 0, 0)),
            pl.BlockSpec((1, AD, A), lambda b: (b, 0, 0)),
            pl.BlockSpec((1, A, 2), lambda b: (b, 0, 0)),
            pl.BlockSpec((1, 2, A), lambda b: (b, 0, 0)),
            pl.BlockSpec((1, AD, A), lambda b: (b, 0, 0)),
            pl.BlockSpec((AD2, 128), lambda b: (0, 0)),
        ],
        out_specs=pl.BlockSpec((1, AD, A), lambda b: (b, 0, 0)),
        out_shape=jax.ShapeDtypeStruct((B, AD, A), logits.dtype),
        interpret=interpret,
    )(lgt, illt, pos32, post32, q0t, c2t)
    return jnp.swapaxes(outt, 1, 2)


# speedup vs baseline: 1201.2758x; 1.5111x over previous
import functools

import jax
import jax.numpy as jnp
from jax import lax
from jax.experimental import pallas as pl
from jax.experimental.pallas import tpu as pltpu
from jax.experimental.pallas import tpu_sc as plsc

ITERATIONS = 3
AD = 5
FOV = 11
MAX_DIST = FOV // 2
NPARAM = FOV * FOV
A = 128
L = 16
NCHUNK = A // L
CTLEN = NPARAM * AD * AD
INVALID = CTLEN
CTPAD = 3056
NW = 32
BPW = 256 // NW


def _sc_body(ct_hbm, lgt_hbm, ill_hbm, q0_hbm, py_hbm, px_hbm, out_hbm,
             ct_v, lgt_v, ill_v, q0_v, qlog_v, qp_v, py_v, px_v, pb_v):
    wid = lax.axis_index("s") * 2 + lax.axis_index("c")
    pltpu.sync_copy(ct_hbm, ct_v)

    def batch_body(bi, carry):
        b = wid * BPW + bi
        pltpu.sync_copy(lgt_hbm.at[b], lgt_v)
        pltpu.sync_copy(ill_hbm.at[b], ill_v)
        pltpu.sync_copy(q0_hbm.at[b], q0_v)
        pltpu.sync_copy(py_hbm.at[b], py_v)
        pltpu.sync_copy(px_hbm.at[b], px_v)

        for a in range(AD):
            for c in range(NCHUNK):
                s = pl.ds(c * L, L)
                qlog_v[a, s] = q0_v[a, s] - ill_v[a, s] * 1e10

        def pb_body(j, c):
            jf = jnp.full((L,), j, jnp.int32)
            yj = plsc.load_gather(py_v, [jf])
            xj = plsc.load_gather(px_v, [jf])
            for ic in range(NCHUNK):
                s = pl.ds(ic * L, L)
                dy = jnp.abs(py_v[s] - yj)
                dx = jnp.abs(px_v[s] - xj)
                nb = jnp.maximum(dy, dx) <= MAX_DIST
                pb = ((dy + MAX_DIST) * FOV + (dx + MAX_DIST)) * (AD * AD)
                pb_v[j, s] = jnp.where(nb, pb, INVALID)
            return c

        lax.fori_loop(0, A, pb_body, 0)

        for _ in range(ITERATIONS):
            for c in range(NCHUNK):
                s = pl.ds(c * L, L)
                qs = [qlog_v[a, s] for a in range(AD)]
                m = qs[0]
                for a in range(1, AD):
                    m = jnp.maximum(m, qs[a])
                es = [jnp.exp(q - m) for q in qs]
                tot = es[0]
                for a in range(1, AD):
                    tot = tot + es[a]
                for a in range(AD):
                    qp_v[a, s] = es[a] / tot

            for ic in range(NCHUNK):
                s = pl.ds(ic * L, L)

                def tail_body(j, accs):
                    accs = list(accs)
                    jf = jnp.full((L,), j, jnp.int32)
                    base = pb_v[j, s]
                    for a2 in range(AD):
                        qsp = plsc.load_gather(
                            qp_v, [jnp.full((L,), a2, jnp.int32), jf])
                        for a in range(AD):
                            cv = plsc.load_gather(ct_v, [base + (a * AD + a2)])
                            accs[a] = accs[a] + cv * qsp
                    return tuple(accs)

                accs = lax.fori_loop(
                    0, A, tail_body,
                    tuple(jnp.zeros((L,), jnp.float32) for _ in range(AD)))
                for a in range(AD):
                    qlog_v[a, s] = (lgt_v[a, s] + accs[a]
                                    - ill_v[a, s] * 1e10)

        pltpu.sync_copy(qlog_v, out_hbm.at[b])
        return carry

    lax.fori_loop(0, BPW, batch_body, 0)


@jax.jit
def kernel(logits, illegal_action_masks, curr_positions, correlation_params):
    B, A_, AD_ = logits.shape
    lgt = jnp.swapaxes(logits, 1, 2)
    illt = jnp.swapaxes(illegal_action_masks, 1, 2)
    q0 = jax.random.uniform(jax.random.key(1), logits.shape,
                            dtype=logits.dtype)
    q0t = jnp.swapaxes(q0, 1, 2)
    pos = curr_positions.astype(jnp.int32)
    py = pos[:, :, 0]
    px = pos[:, :, 1]
    ct = jnp.pad(correlation_params.reshape(-1), (0, CTPAD - CTLEN))

    mesh = plsc.VectorSubcoreMesh(core_axis_name="c", subcore_axis_name="s")
    run = functools.partial(
        pl.kernel,
        mesh=mesh,
        compiler_params=pltpu.CompilerParams(needs_layout_passes=False),
        out_type=jax.ShapeDtypeStruct((B, AD_, A_), jnp.float32),
        scratch_types=[
            pltpu.VMEM((CTPAD,), jnp.float32),
            pltpu.VMEM((AD, A), jnp.float32),
            pltpu.VMEM((AD, A), jnp.float32),
            pltpu.VMEM((AD, A), jnp.float32),
            pltpu.VMEM((AD, A), jnp.float32),
            pltpu.VMEM((AD, A), jnp.float32),
            pltpu.VMEM((A,), jnp.int32),
            pltpu.VMEM((A,), jnp.int32),
            pltpu.VMEM((A, A), jnp.int32),
        ],
    )(_sc_body)
    outt = run(ct, lgt, illt, q0t, py, px)
    return jnp.swapaxes(outt, 1, 2)


# speedup vs baseline: 1568.2411x; 1.3055x over previous
import functools

import jax
import jax.numpy as jnp
from jax import lax
from jax.experimental import pallas as pl
from jax.experimental.pallas import tpu as pltpu
from jax.experimental.pallas import tpu_sc as plsc

ITERATIONS = 3
AD = 5
FOV = 11
MAX_DIST = FOV // 2
NPARAM = FOV * FOV
A = 128
L = 16
NCHUNK = A // L
AP = A + L
CTLEN = NPARAM * AD * AD
INVALID = CTLEN
CTPAD = 3056
NW = 32
BPW = 256 // NW


def _sc_body(ct_hbm, lgt_hbm, ill_hbm, q0_hbm, py_hbm, px_hbm, out_hbm,
             ct_v, lgt_v, ill_v, q0_v, qlog_v, qp_v, corr_v, py_v, px_v,
             pb_v, hd_v, cnt_v):
    wid = lax.axis_index("s") * 2 + lax.axis_index("c")
    pltpu.sync_copy(ct_hbm, ct_v)
    lanes = lax.iota(jnp.int32, L)
    lane0 = lanes == 0

    def batch_body(bi, carry):
        b = wid * BPW + bi
        pltpu.sync_copy(lgt_hbm.at[b], lgt_v)
        pltpu.sync_copy(ill_hbm.at[b], ill_v)
        pltpu.sync_copy(q0_hbm.at[b], q0_v)
        pltpu.sync_copy(py_hbm.at[b], py_v)
        pltpu.sync_copy(px_hbm.at[b], px_v)

        for a in range(AD):
            for c in range(NCHUNK):
                s = pl.ds(c * L, L)
                qlog_v[a, s] = q0_v[a, s] - ill_v[a, s] * 1e10

        def build_body(j, c):
            jf = jnp.full((L,), j, jnp.int32)
            yj = plsc.load_gather(py_v, [jf])
            xj = plsc.load_gather(px_v, [jf])
            for ic in range(NCHUNK):
                hd_v[j, pl.ds(ic * L, L)] = A + lanes
            cntv = jnp.zeros((L,), jnp.int32)
            for ic in range(NCHUNK):
                s = pl.ds(ic * L, L)
                dy = jnp.abs(py_v[s] - yj)
                dx = jnp.abs(px_v[s] - xj)
                nb = jnp.maximum(dy, dx) <= MAX_DIST
                pb = ((dy + MAX_DIST) * FOV + (dx + MAX_DIST)) * (AD * AD)
                pb_v[j, s] = jnp.where(nb, pb, INVALID)
                pos = cntv + jnp.cumsum(nb.astype(jnp.int32)) - 1
                plsc.store_scatter(hd_v, [jf, pos], ic * L + lanes, mask=nb)
                cntv = cntv + plsc.all_reduce_population_count(nb)
            pb_v[j, pl.ds(A, L)] = jnp.full((L,), INVALID, jnp.int32)
            plsc.store_scatter(cnt_v, [jf], cntv, mask=lane0)
            return c

        lax.fori_loop(0, A, build_body, 0)

        for _ in range(ITERATIONS):
            for c in range(NCHUNK):
                s = pl.ds(c * L, L)
                qs = [qlog_v[a, s] for a in range(AD)]
                m = qs[0]
                for a in range(1, AD):
                    m = jnp.maximum(m, qs[a])
                es = [jnp.exp(q - m) for q in qs]
                tot = es[0]
                for a in range(1, AD):
                    tot = tot + es[a]
                for a in range(AD):
                    qp_v[a, s] = es[a] / tot

            for a in range(AD):
                for c in range(NCHUNK + 1):
                    corr_v[a, pl.ds(c * L, L)] = jnp.zeros((L,), jnp.float32)

            def tail_body(j, c):
                jf = jnp.full((L,), j, jnp.int32)
                qsp = [plsc.load_gather(
                    qp_v, [jnp.full((L,), a2, jnp.int32), jf])
                    for a2 in range(AD)]
                cntv = plsc.load_gather(cnt_v, [jf])
                nch = (cntv[0] + (L - 1)) // L

                def chunk_body(ci, cc):
                    hv = plsc.load_gather(hd_v, [jf, ci * L + lanes])
                    basev = plsc.load_gather(pb_v, [jf, hv])
                    accs = [jnp.zeros((L,), jnp.float32) for _ in range(AD)]
                    for a2 in range(AD):
                        for a in range(AD):
                            cv = plsc.load_gather(
                                ct_v, [basev + (a * AD + a2)])
                            accs[a] = accs[a] + cv * qsp[a2]
                    for a in range(AD):
                        plsc.addupdate_scatter(
                            corr_v, [jnp.full((L,), a, jnp.int32), hv],
                            accs[a])
                    return cc

                lax.fori_loop(0, nch, chunk_body, 0)
                return c

            lax.fori_loop(0, A, tail_body, 0)

            for a in range(AD):
                for c in range(NCHUNK):
                    s = pl.ds(c * L, L)
                    qlog_v[a, s] = (lgt_v[a, s] + corr_v[a, s]
                                    - ill_v[a, s] * 1e10)

        pltpu.sync_copy(qlog_v, out_hbm.at[b])
        return carry

    lax.fori_loop(0, BPW, batch_body, 0)


@jax.jit
def kernel(logits, illegal_action_masks, curr_positions, correlation_params):
    B, A_, AD_ = logits.shape
    lgt = jnp.swapaxes(logits, 1, 2)
    illt = jnp.swapaxes(illegal_action_masks, 1, 2)
    q0 = jax.random.uniform(jax.random.key(1), logits.shape,
                            dtype=logits.dtype)
    q0t = jnp.swapaxes(q0, 1, 2)
    pos = curr_positions.astype(jnp.int32)
    py = pos[:, :, 0]
    px = pos[:, :, 1]
    ct = jnp.pad(correlation_params.reshape(-1), (0, CTPAD - CTLEN))

    mesh = plsc.VectorSubcoreMesh(core_axis_name="c", subcore_axis_name="s")
    run = functools.partial(
        pl.kernel,
        mesh=mesh,
        compiler_params=pltpu.CompilerParams(needs_layout_passes=False),
        out_type=jax.ShapeDtypeStruct((B, AD_, A_), jnp.float32),
        scratch_types=[
            pltpu.VMEM((CTPAD,), jnp.float32),
            pltpu.VMEM((AD, A), jnp.float32),
            pltpu.VMEM((AD, A), jnp.float32),
            pltpu.VMEM((AD, A), jnp.float32),
            pltpu.VMEM((AD, A), jnp.float32),
            pltpu.VMEM((AD, A), jnp.float32),
            pltpu.VMEM((AD, AP), jnp.float32),
            pltpu.VMEM((A,), jnp.int32),
            pltpu.VMEM((A,), jnp.int32),
            pltpu.VMEM((A, AP), jnp.int32),
            pltpu.VMEM((A, A), jnp.int32),
            pltpu.VMEM((A,), jnp.int32),
        ],
    )(_sc_body)
    outt = run(ct, lgt, illt, q0t, py, px)
    return jnp.swapaxes(outt, 1, 2)


# speedup vs baseline: 1636.0677x; 1.0433x over previous
import functools

import jax
import jax.numpy as jnp
from jax import lax
from jax.experimental import pallas as pl
from jax.experimental.pallas import tpu as pltpu
from jax.experimental.pallas import tpu_sc as plsc

ITERATIONS = 3
AD = 5
FOV = 11
MAX_DIST = FOV // 2
NPARAM = FOV * FOV
A = 128
L = 16
NCHUNK = A // L
AP = A + L
CTLEN = NPARAM * AD * AD
INVALID = CTLEN
CTPAD = 3056
NW = 32
BPW = 256 // NW


def _sc_body(ct_hbm, lgt_hbm, ill_hbm, q0_hbm, py_hbm, px_hbm, out_hbm,
             ct_v, lgt_v, ill_v, q0_v, qlog_v, qp_v, corr_v, py_v, px_v,
             pb_v, hd_v, cnt_v):
    wid = lax.axis_index("s") * 2 + lax.axis_index("c")
    pltpu.sync_copy(ct_hbm, ct_v)
    lanes = lax.iota(jnp.int32, L)
    lane0 = lanes == 0

    def batch_body(bi, carry):
        b = wid * BPW + bi
        pltpu.sync_copy(lgt_hbm.at[b], lgt_v)
        pltpu.sync_copy(ill_hbm.at[b], ill_v)
        pltpu.sync_copy(q0_hbm.at[b], q0_v)
        pltpu.sync_copy(py_hbm.at[b], py_v)
        pltpu.sync_copy(px_hbm.at[b], px_v)

        for a in range(AD):
            for c in range(NCHUNK):
                s = pl.ds(c * L, L)
                qlog_v[a, s] = q0_v[a, s] - ill_v[a, s] * 1e10

        def build_body(j, c):
            jf = jnp.full((L,), j, jnp.int32)
            yj = plsc.load_gather(py_v, [jf])
            xj = plsc.load_gather(px_v, [jf])
            for ic in range(NCHUNK):
                hd_v[j, pl.ds(ic * L, L)] = A + lanes
            cntv = jnp.zeros((L,), jnp.int32)
            for ic in range(NCHUNK):
                s = pl.ds(ic * L, L)
                dy = jnp.abs(py_v[s] - yj)
                dx = jnp.abs(px_v[s] - xj)
                nb = jnp.maximum(dy, dx) <= MAX_DIST
                pb = ((dy + MAX_DIST) * FOV + (dx + MAX_DIST)) * (AD * AD)
                pb_v[j, s] = jnp.where(nb, pb, INVALID)
                pos = cntv + jnp.cumsum(nb.astype(jnp.int32)) - 1
                plsc.store_scatter(hd_v, [jf, pos], ic * L + lanes, mask=nb)
                cntv = cntv + plsc.all_reduce_population_count(nb)
            pb_v[j, pl.ds(A, L)] = jnp.full((L,), INVALID, jnp.int32)
            nchv = (cntv + (L - 1)) // L
            plsc.store_scatter(cnt_v, [jf], nchv, mask=lane0)
            return c

        lax.fori_loop(0, A, build_body, 0)

        for _ in range(ITERATIONS):
            for c in range(NCHUNK):
                s = pl.ds(c * L, L)
                qs = [qlog_v[a, s] for a in range(AD)]
                m = qs[0]
                for a in range(1, AD):
                    m = jnp.maximum(m, qs[a])
                es = [jnp.exp(q - m) for q in qs]
                tot = es[0]
                for a in range(1, AD):
                    tot = tot + es[a]
                for a in range(AD):
                    qp_v[a, s] = es[a] / tot

            for a in range(AD):
                for c in range(NCHUNK + 1):
                    corr_v[a, pl.ds(c * L, L)] = jnp.zeros((L,), jnp.float32)

            def tail_body(j, c):
                jf = jnp.full((L,), j, jnp.int32)
                qsp = [plsc.load_gather(
                    qp_v, [jnp.full((L,), a2, jnp.int32), jf])
                    for a2 in range(AD)]
                nch = plsc.load_gather(cnt_v, [jf])[0]

                def chunk_body(ci, cc):
                    hv = plsc.load_gather(hd_v, [jf, ci * L + lanes])
                    basev = plsc.load_gather(pb_v, [jf, hv])
                    accs = [jnp.zeros((L,), jnp.float32) for _ in range(AD)]
                    for a2 in range(AD):
                        for a in range(AD):
                            cv = plsc.load_gather(
                                ct_v, [basev + (a * AD + a2)])
                            accs[a] = accs[a] + cv * qsp[a2]
                    for a in range(AD):
                        plsc.addupdate_scatter(
                            corr_v, [jnp.full((L,), a, jnp.int32), hv],
                            accs[a])
                    return cc

                lax.fori_loop(0, nch, chunk_body, 0)
                return c

            lax.fori_loop(0, A, tail_body, 0, unroll=4)

            for a in range(AD):
                for c in range(NCHUNK):
                    s = pl.ds(c * L, L)
                    qlog_v[a, s] = (lgt_v[a, s] + corr_v[a, s]
                                    - ill_v[a, s] * 1e10)

        pltpu.sync_copy(qlog_v, out_hbm.at[b])
        return carry

    lax.fori_loop(0, BPW, batch_body, 0)


@jax.jit
def kernel(logits, illegal_action_masks, curr_positions, correlation_params):
    B, A_, AD_ = logits.shape
    lgt = jnp.swapaxes(logits, 1, 2)
    illt = jnp.swapaxes(illegal_action_masks, 1, 2)
    q0 = jax.random.uniform(jax.random.key(1), logits.shape,
                            dtype=logits.dtype)
    q0t = jnp.swapaxes(q0, 1, 2)
    pos = curr_positions.astype(jnp.int32)
    py = pos[:, :, 0]
    px = pos[:, :, 1]
    ct = jnp.pad(correlation_params.reshape(-1), (0, CTPAD - CTLEN))

    mesh = plsc.VectorSubcoreMesh(core_axis_name="c", subcore_axis_name="s")
    run = functools.partial(
        pl.kernel,
        mesh=mesh,
        compiler_params=pltpu.CompilerParams(needs_layout_passes=False),
        out_type=jax.ShapeDtypeStruct((B, AD_, A_), jnp.float32),
        scratch_types=[
            pltpu.VMEM((CTPAD,), jnp.float32),
            pltpu.VMEM((AD, A), jnp.float32),
            pltpu.VMEM((AD, A), jnp.float32),
            pltpu.VMEM((AD, A), jnp.float32),
            pltpu.VMEM((AD, A), jnp.float32),
            pltpu.VMEM((AD, A), jnp.float32),
            pltpu.VMEM((AD, AP), jnp.float32),
            pltpu.VMEM((A,), jnp.int32),
            pltpu.VMEM((A,), jnp.int32),
            pltpu.VMEM((A, AP), jnp.int32),
            pltpu.VMEM((A, A), jnp.int32),
            pltpu.VMEM((A,), jnp.int32),
        ],
    )(_sc_body)
    outt = run(ct, lgt, illt, q0t, py, px)
    return jnp.swapaxes(outt, 1, 2)


# speedup vs baseline: 1699.1990x; 1.0386x over previous
import functools

import jax
import jax.numpy as jnp
from jax import lax
from jax.experimental import pallas as pl
from jax.experimental.pallas import tpu as pltpu
from jax.experimental.pallas import tpu_sc as plsc

ITERATIONS = 3
AD = 5
FOV = 11
MAX_DIST = FOV // 2
NPARAM = FOV * FOV
A = 128
L = 16
NCHUNK = A // L
AP = A + L
CTLEN = NPARAM * AD * AD
INVALID = CTLEN
CTPAD = 3056
MAXC = A * NCHUNK
NW = 32
BPW = 256 // NW


def _sc_body(ct_hbm, lgt_hbm, ill_hbm, q0_hbm, py_hbm, px_hbm, out_hbm,
             ct_v, lgt_v, ill_v, q0_v, qlog_v, qp_v, corr_v, py_v, px_v,
             pb_v, hd_v, ctail_v, cslot_v):
    wid = lax.axis_index("s") * 2 + lax.axis_index("c")
    pltpu.sync_copy(ct_hbm, ct_v)
    lanes = lax.iota(jnp.int32, L)

    def batch_body(bi, carry):
        b = wid * BPW + bi
        pltpu.sync_copy(lgt_hbm.at[b], lgt_v)
        pltpu.sync_copy(ill_hbm.at[b], ill_v)
        pltpu.sync_copy(q0_hbm.at[b], q0_v)
        pltpu.sync_copy(py_hbm.at[b], py_v)
        pltpu.sync_copy(px_hbm.at[b], px_v)

        for a in range(AD):
            for c in range(NCHUNK):
                s = pl.ds(c * L, L)
                qlog_v[a, s] = q0_v[a, s] - ill_v[a, s] * 1e10

        for c in range(MAXC // L):
            s = pl.ds(c * L, L)
            ctail_v[s] = jnp.zeros((L,), jnp.int32)
            cslot_v[s] = jnp.full((L,), NCHUNK, jnp.int32)

        def build_body(j, mv):
            jf = jnp.full((L,), j, jnp.int32)
            yj = plsc.load_gather(py_v, [jf])
            xj = plsc.load_gather(px_v, [jf])
            for ic in range(NCHUNK + 1):
                hd_v[j, pl.ds(ic * L, L)] = A + lanes
            cntv = jnp.zeros((L,), jnp.int32)
            for ic in range(NCHUNK):
                s = pl.ds(ic * L, L)
                dy = jnp.abs(py_v[s] - yj)
                dx = jnp.abs(px_v[s] - xj)
                nb = jnp.maximum(dy, dx) <= MAX_DIST
                pb = ((dy + MAX_DIST) * FOV + (dx + MAX_DIST)) * (AD * AD)
                pb_v[j, s] = jnp.where(nb, pb, INVALID)
                pos = cntv + jnp.cumsum(nb.astype(jnp.int32)) - 1
                plsc.store_scatter(hd_v, [jf, pos], ic * L + lanes, mask=nb)
                cntv = cntv + plsc.all_reduce_population_count(nb)
            pb_v[j, pl.ds(A, L)] = jnp.full((L,), INVALID, jnp.int32)
            nchv = (cntv + (L - 1)) // L
            plsc.store_scatter(ctail_v, [mv + lanes], jf, mask=lanes < nchv)
            plsc.store_scatter(cslot_v, [mv + lanes], lanes, mask=lanes < nchv)
            return mv + nchv

        mv = lax.fori_loop(0, A, build_body, jnp.zeros((L,), jnp.int32))
        nhalf = (mv[0] + 1) // 2

        for _ in range(ITERATIONS):
            for c in range(NCHUNK):
                s = pl.ds(c * L, L)
                qs = [qlog_v[a, s] for a in range(AD)]
                m = qs[0]
                for a in range(1, AD):
                    m = jnp.maximum(m, qs[a])
                es = [jnp.exp(q - m) for q in qs]
                tot = es[0]
                for a in range(1, AD):
                    tot = tot + es[a]
                for a in range(AD):
                    qp_v[a, s] = es[a] / tot

            for a in range(AD):
                for c in range(NCHUNK + 1):
                    corr_v[a, pl.ds(c * L, L)] = jnp.zeros((L,), jnp.float32)

            def chunk_body(ci, cc):
                for e in range(2):
                    ef = jnp.full((L,), 2 * ci + e, jnp.int32)
                    jf = plsc.load_gather(ctail_v, [ef])
                    slot = plsc.load_gather(cslot_v, [ef])
                    hv = plsc.load_gather(hd_v, [jf, slot * L + lanes])
                    basev = plsc.load_gather(pb_v, [jf, hv])
                    qsp = [plsc.load_gather(
                        qp_v, [jnp.full((L,), a2, jnp.int32), jf])
                        for a2 in range(AD)]
                    accs = [jnp.zeros((L,), jnp.float32) for _ in range(AD)]
                    for a2 in range(AD):
                        for a in range(AD):
                            cv = plsc.load_gather(
                                ct_v, [basev + (a * AD + a2)])
                            accs[a] = accs[a] + cv * qsp[a2]
                    for a in range(AD):
                        plsc.addupdate_scatter(
                            corr_v, [jnp.full((L,), a, jnp.int32), hv],
                            accs[a])
                return cc

            lax.fori_loop(0, nhalf, chunk_body, 0)

            for a in range(AD):
                for c in range(NCHUNK):
                    s = pl.ds(c * L, L)
                    qlog_v[a, s] = (lgt_v[a, s] + corr_v[a, s]
                                    - ill_v[a, s] * 1e10)

        pltpu.sync_copy(qlog_v, out_hbm.at[b])
        return carry

    lax.fori_loop(0, BPW, batch_body, 0)


@jax.jit
def kernel(logits, illegal_action_masks, curr_positions, correlation_params):
    B, A_, AD_ = logits.shape
    lgt = jnp.swapaxes(logits, 1, 2)
    illt = jnp.swapaxes(illegal_action_masks, 1, 2)
    q0 = jax.random.uniform(jax.random.key(1), logits.shape,
                            dtype=logits.dtype)
    q0t = jnp.swapaxes(q0, 1, 2)
    pos = curr_positions.astype(jnp.int32)
    py = pos[:, :, 0]
    px = pos[:, :, 1]
    ct = jnp.pad(correlation_params.reshape(-1), (0, CTPAD - CTLEN))

    mesh = plsc.VectorSubcoreMesh(core_axis_name="c", subcore_axis_name="s")
    run = functools.partial(
        pl.kernel,
        mesh=mesh,
        compiler_params=pltpu.CompilerParams(needs_layout_passes=False),
        out_type=jax.ShapeDtypeStruct((B, AD_, A_), jnp.float32),
        scratch_types=[
            pltpu.VMEM((CTPAD,), jnp.float32),
            pltpu.VMEM((AD, A), jnp.float32),
            pltpu.VMEM((AD, A), jnp.float32),
            pltpu.VMEM((AD, A), jnp.float32),
            pltpu.VMEM((AD, A), jnp.float32),
            pltpu.VMEM((AD, A), jnp.float32),
            pltpu.VMEM((AD, AP), jnp.float32),
            pltpu.VMEM((A,), jnp.int32),
            pltpu.VMEM((A,), jnp.int32),
            pltpu.VMEM((A, AP), jnp.int32),
            pltpu.VMEM((A, AP), jnp.int32),
            pltpu.VMEM((MAXC,), jnp.int32),
            pltpu.VMEM((MAXC,), jnp.int32),
        ],
    )(_sc_body)
    outt = run(ct, lgt, illt, q0t, py, px)
    return jnp.swapaxes(outt, 1, 2)
